# baseline (speedup 1.0000x reference)
import functools

import jax
import jax.numpy as jnp
from jax import lax
from jax.experimental import pallas as pl
from jax.experimental.pallas import tpu as pltpu
from jax.experimental.pallas import tpu_sc as plsc

_D_IN = 768
_D_SAE = 12288
_B = 8192
_K = 32
_K_TOTAL = _K * _B
_N_TOTAL = _B * _D_SAE


_BM_E = 1024
_BN_E = 1536


def _enc_body(x_ref, w_ref, be_ref, pre_ref):
    acc = jnp.dot(x_ref[...], w_ref[...], preferred_element_type=jnp.float32)
    pre_ref[...] = jnp.maximum(acc + be_ref[...], 0.0)


def _encode(xcb, W_encb, b_enc2d):
    half = _B // 2
    grid = (_D_SAE // _BN_E, half // _BM_E)
    return pl.pallas_call(
        _enc_body,
        grid=grid,
        in_specs=[
            pl.BlockSpec((_BM_E, _D_IN), lambda j, i: (i, 0)),
            pl.BlockSpec((_D_IN, _BN_E), lambda j, i: (0, j)),
            pl.BlockSpec((1, _BN_E), lambda j, i: (0, j)),
        ],
        out_specs=pl.BlockSpec((_BM_E, _BN_E), lambda j, i: (i, j)),
        out_shape=jax.ShapeDtypeStruct((half, _D_SAE), jnp.float32),
    )(xcb, W_encb, b_enc2d)


_NTILES = 32
_ROWS_PT = _B // _NTILES
_NB1 = 32768
_NB2 = 32768
_UNROLL = 16

def _tile_wid():
    return lax.axis_index("s") * 2 + lax.axis_index("c")


def _zero_hist(hist_ref, nb):
    zeros = jnp.zeros((16,), jnp.int32)

    @plsc.parallel_loop(0, nb // 16, 1, unroll=8)
    def _(i):
        hist_ref[pl.ds(i * 16, 16)] = zeros


def _sc_mesh():
    return plsc.VectorSubcoreMesh(
        core_axis_name="c", subcore_axis_name="s", num_cores=2, num_subcores=16
    )


_ROWS_HALF = _B // 2 // _NTILES


def _make_process(ha, hb, binfn):
    ones = jnp.ones((16,), jnp.int32)
    nv = _D_SAE // 16

    def process(buf):
        @plsc.parallel_loop(0, nv, 2, unroll=_UNROLL // 2)
        def _(i):
            v0 = buf[pl.ds(i * 16, 16)]
            bins0, m0 = binfn(plsc.bitcast(v0, jnp.int32))
            plsc.addupdate_scatter(ha, [bins0], ones, mask=m0)
            v1 = buf[pl.ds((i + 1) * 16, 16)]
            bins1, m1 = binfn(plsc.bitcast(v1, jnp.int32))
            plsc.addupdate_scatter(hb, [bins1], ones, mask=m1)

    return process


def _stream_rows(ref, base, nrows, b0, b1, s0, s1, process):
    pltpu.async_copy(ref.at[base], b0, s0)

    def pair(c, _):
        r = base + 2 * c
        pltpu.async_copy(ref.at[r + 1], b1, s1)
        pltpu.make_async_copy(ref.at[r], b0, s0).wait()
        process(b0)

        @pl.when(2 * c + 2 < nrows)
        def _():
            pltpu.async_copy(ref.at[r + 2], b0, s0)

        pltpu.make_async_copy(ref.at[r + 1], b1, s1).wait()
        process(b1)
        return 0

    lax.fori_loop(0, nrows // 2, pair, 0)


@functools.lru_cache(maxsize=None)
def _make_hist_hi():
    @functools.partial(
        pl.kernel,
        out_type=jax.ShapeDtypeStruct((2 * _NTILES, _NB1), jnp.int32),
        mesh=_sc_mesh(),
        compiler_params=pltpu.CompilerParams(needs_layout_passes=False),
        scratch_types=[
            pltpu.VMEM((_D_SAE,), jnp.float32),
            pltpu.VMEM((_D_SAE,), jnp.float32),
            pltpu.VMEM((_NB1,), jnp.int32),
            pltpu.VMEM((_NB1,), jnp.int32),
            pltpu.SemaphoreType.DMA,
            pltpu.SemaphoreType.DMA,
        ],
    )
    def _hist_hi(pre_hbm, out_hbm, b0, b1, ha, hb, s0, s1):
        wid = _tile_wid()
        _zero_hist(ha, _NB1)
        _zero_hist(hb, _NB1)

        def binfn(bits):
            hi = lax.shift_right_logical(bits, 16)
            return hi, hi > 0

        process = _make_process(ha, hb, binfn)
        _stream_rows(pre_hbm, wid * _ROWS_HALF, _ROWS_HALF, b0, b1, s0, s1,
                     process)
        pltpu.sync_copy(ha, out_hbm.at[2 * wid])
        pltpu.sync_copy(hb, out_hbm.at[2 * wid + 1])

    return _hist_hi


@functools.lru_cache(maxsize=None)
def _make_hist_lo():
    @functools.partial(
        pl.kernel,
        out_type=jax.ShapeDtypeStruct((2 * _NTILES, _NB2), jnp.int32),
        mesh=_sc_mesh(),
        compiler_params=pltpu.CompilerParams(needs_layout_passes=False),
        scratch_types=[
            pltpu.VMEM((_D_SAE,), jnp.float32),
            pltpu.VMEM((_D_SAE,), jnp.float32),
            pltpu.VMEM((_NB2,), jnp.int32),
            pltpu.VMEM((_NB2,), jnp.int32),
            pltpu.VMEM((16,), jnp.int32),
            pltpu.SemaphoreType.DMA,
            pltpu.SemaphoreType.DMA,
        ],
    )
    def _hist_lo(prea_hbm, preb_hbm, sel_hbm, out_hbm, b0, b1, ha, hb, selv,
                 s0, s1):
        wid = _tile_wid()
        _zero_hist(ha, _NB2)
        _zero_hist(hb, _NB2)
        pltpu.sync_copy(sel_hbm, selv)
        sel = selv[...]
        m15 = jnp.full((16,), 0x7FFF, jnp.int32)

        def binfn(bits):
            hi = lax.shift_right_logical(bits, 16)
            mid = jnp.bitwise_and(lax.shift_right_logical(bits, 1), m15)
            return mid, hi == sel

        process = _make_process(ha, hb, binfn)
        base = wid * _ROWS_HALF
        _stream_rows(prea_hbm, base, _ROWS_HALF, b0, b1, s0, s1, process)
        _stream_rows(preb_hbm, base, _ROWS_HALF, b0, b1, s0, s1, process)
        pltpu.sync_copy(ha, out_hbm.at[2 * wid])
        pltpu.sync_copy(hb, out_hbm.at[2 * wid + 1])

    return _hist_lo


def _scan_body(k_ref, h_ref, b_ref, a_ref):
    h = jnp.sum(h_ref[...], axis=0)
    hf = h.astype(jnp.float32)
    r, c = h.shape
    ci = lax.broadcasted_iota(jnp.int32, (c, c), 0)
    cj = lax.broadcasted_iota(jnp.int32, (c, c), 1)
    tri = (ci >= cj).astype(jnp.float32)
    rs = jax.lax.dot(hf, tri, precision=jax.lax.Precision.HIGHEST)
    ri = lax.broadcasted_iota(jnp.int32, (r, r), 0)
    rj = lax.broadcasted_iota(jnp.int32, (r, r), 1)
    upp = (rj > ri).astype(jnp.float32)
    t = rs[:, 0:1]
    sr = jax.lax.dot(upp, t, precision=jax.lax.Precision.HIGHEST)
    s = rs + sr
    kf = k_ref[0, 0].astype(jnp.float32)
    nkeep = jnp.sum((s >= kf).astype(jnp.int32))
    bstar = nkeep - 1
    flat = (lax.broadcasted_iota(jnp.int32, (r, c), 0) * c
            + lax.broadcasted_iota(jnp.int32, (r, c), 1))
    above = jnp.sum(jnp.where(flat > bstar, h, 0))
    b_ref[0, 0] = bstar
    a_ref[0, 0] = above


def _scan(hists3d, k11):
    nt, r, c = hists3d.shape
    return pl.pallas_call(
        _scan_body,
        in_specs=[
            pl.BlockSpec(memory_space=pltpu.SMEM),
            pl.BlockSpec((nt, r, c), lambda: (0, 0, 0)),
        ],
        out_specs=[
            pl.BlockSpec(memory_space=pltpu.SMEM),
            pl.BlockSpec(memory_space=pltpu.SMEM),
        ],
        out_shape=[
            jax.ShapeDtypeStruct((1, 1), jnp.int32),
            jax.ShapeDtypeStruct((1, 1), jnp.int32),
        ],
    )(k11, hists3d)


_BM_D = 1024
_BK_D = 1536


def _dec_body(tau_ref, pre_ref, wd_ref, bd_ref, z_ref, rec_ref):
    j = pl.program_id(1)
    tau = tau_ref[0, 0]
    p = pre_ref[...]
    z = jnp.where(p >= tau, p, 0.0)
    z_ref[...] = z
    zb = z.astype(jnp.bfloat16)
    contrib = jnp.dot(zb, wd_ref[...], preferred_element_type=jnp.float32)

    @pl.when(j == 0)
    def _():
        rec_ref[...] = contrib + bd_ref[...]

    @pl.when(j > 0)
    def _():
        rec_ref[...] += contrib


def _dec_body_b(z_prev_ref, tau_ref, pre_ref, wd_ref, bd_ref, z_ref, rec_ref):
    del z_prev_ref
    _dec_body(tau_ref, pre_ref, wd_ref, bd_ref, z_ref, rec_ref)


def _decode_half(tau11, pre_half, W_decb, b_dec2d, half_idx, z_prev=None):
    half = _B // 2
    grid = (half // _BM_D, _D_SAE // _BK_D)
    roff = half_idx * (half // _BM_D)
    in_specs = [
        pl.BlockSpec(memory_space=pltpu.SMEM),
        pl.BlockSpec((_BM_D, _BK_D), lambda i, j: (i, j)),
        pl.BlockSpec((_BK_D, _D_IN), lambda i, j: (j, 0)),
        pl.BlockSpec((1, _D_IN), lambda i, j: (0, 0)),
    ]
    args = (tau11, pre_half, W_decb, b_dec2d)
    body = _dec_body
    aliases = {}
    if z_prev is not None:
        in_specs = [pl.BlockSpec(memory_space=pl.ANY)] + in_specs
        args = (z_prev,) + args
        body = _dec_body_b
        aliases = {0: 0}
    return pl.pallas_call(
        body,
        grid=grid,
        in_specs=in_specs,
        out_specs=[
            pl.BlockSpec((_BM_D, _BK_D), lambda i, j, r=roff: (i + r, j)),
            pl.BlockSpec((_BM_D, _D_IN), lambda i, j: (i, 0)),
        ],
        out_shape=[
            jax.ShapeDtypeStruct((_B, _D_SAE), jnp.float32),
            jax.ShapeDtypeStruct((half, _D_IN), jnp.float32),
        ],
        input_output_aliases=aliases,
    )(*args)


def kernel(x, W_enc, b_enc, W_dec, b_dec):
    xcb = (x - b_dec[None, :]).astype(jnp.bfloat16)
    W_encb = W_enc.astype(jnp.bfloat16)
    W_decb = W_dec.astype(jnp.bfloat16)
    b_enc2d = b_enc.reshape(1, _D_SAE)
    b_dec2d = b_dec.reshape(1, _D_IN)

    half = _B // 2
    pre_a = _encode(xcb[:half], W_encb, b_enc2d)
    h1a = _make_hist_hi()(pre_a)
    pre_b = _encode(xcb[half:], W_encb, b_enc2d)
    h1b = _make_hist_hi()(pre_b)

    k11 = jnp.full((1, 1), _K_TOTAL, jnp.int32)
    h1 = jnp.concatenate([h1a, h1b], axis=0)
    b1, a1 = _scan(h1.reshape(4 * _NTILES, _NB1 // 128, 128), k11)

    sel16 = jnp.broadcast_to(b1.reshape(1), (16,)).astype(jnp.int32)
    h2 = _make_hist_lo()(pre_a, pre_b, sel16)
    r11 = k11 - a1
    b2, _ = _scan(h2.reshape(2 * _NTILES, _NB2 // 128, 128), r11)

    tau_bits = jnp.bitwise_or(jnp.left_shift(b1, 16), jnp.left_shift(b2, 1))
    tau11 = jax.lax.bitcast_convert_type(tau_bits, jnp.float32)

    z_a, rec_a = _decode_half(tau11, pre_a, W_decb, b_dec2d, 0)
    z, rec_b = _decode_half(tau11, pre_b, W_decb, b_dec2d, 1, z_prev=z_a)
    recon = jnp.concatenate([rec_a, rec_b], axis=0)
    return (recon, z)


# speedup vs baseline: 1.0068x; 1.0068x over previous
import functools

import jax
import jax.numpy as jnp
from jax import lax
from jax.experimental import pallas as pl
from jax.experimental.pallas import tpu as pltpu
from jax.experimental.pallas import tpu_sc as plsc

_D_IN = 768
_D_SAE = 12288
_B = 8192
_K = 32
_K_TOTAL = _K * _B
_N_TOTAL = _B * _D_SAE


_BM_E = 1024
_BN_E = 1536


def _enc_body(x_ref, w_ref, be_ref, pre_ref):
    acc = jnp.dot(x_ref[...], w_ref[...], preferred_element_type=jnp.float32)
    pre_ref[...] = jnp.maximum(acc + be_ref[...], 0.0)


def _encode(xcb, W_encb, b_enc2d):
    grid = (_D_SAE // _BN_E, _B // _BM_E)
    return pl.pallas_call(
        _enc_body,
        grid=grid,
        in_specs=[
            pl.BlockSpec((_BM_E, _D_IN), lambda j, i: (i, 0)),
            pl.BlockSpec((_D_IN, _BN_E), lambda j, i: (0, j)),
            pl.BlockSpec((1, _BN_E), lambda j, i: (0, j)),
        ],
        out_specs=pl.BlockSpec((_BM_E, _BN_E), lambda j, i: (i, j)),
        out_shape=jax.ShapeDtypeStruct((_B, _D_SAE), jnp.float32),
    )(xcb, W_encb, b_enc2d)


_NTILES = 32
_ROWS_PT = _B // _NTILES
_NB1 = 32768
_NB2 = 32768
_UNROLL = 16

def _tile_wid():
    return lax.axis_index("s") * 2 + lax.axis_index("c")


def _zero_hist(hist_ref, nb):
    zeros = jnp.zeros((16,), jnp.int32)

    @plsc.parallel_loop(0, nb // 16, 1, unroll=8)
    def _(i):
        hist_ref[pl.ds(i * 16, 16)] = zeros


def _sc_mesh():
    return plsc.VectorSubcoreMesh(
        core_axis_name="c", subcore_axis_name="s", num_cores=2, num_subcores=16
    )


def _hist_pass(pre_hbm, out_hbm, b0, b1, ha, hb, s0, s1, wid, binfn):
    base = wid * _ROWS_PT
    _zero_hist(ha, _NB1)
    _zero_hist(hb, _NB1)
    ones = jnp.ones((16,), jnp.int32)
    nv = _D_SAE // 16

    def process(buf):
        @plsc.parallel_loop(0, nv, 2, unroll=_UNROLL // 2)
        def _(i):
            v0 = buf[pl.ds(i * 16, 16)]
            bins0, m0 = binfn(plsc.bitcast(v0, jnp.int32))
            plsc.addupdate_scatter(ha, [bins0], ones, mask=m0)
            v1 = buf[pl.ds((i + 1) * 16, 16)]
            bins1, m1 = binfn(plsc.bitcast(v1, jnp.int32))
            plsc.addupdate_scatter(hb, [bins1], ones, mask=m1)

    pltpu.async_copy(pre_hbm.at[base], b0, s0)

    def pair(c, _):
        r = base + 2 * c
        pltpu.async_copy(pre_hbm.at[r + 1], b1, s1)
        pltpu.make_async_copy(pre_hbm.at[r], b0, s0).wait()
        process(b0)

        @pl.when(2 * c + 2 < _ROWS_PT)
        def _():
            pltpu.async_copy(pre_hbm.at[r + 2], b0, s0)

        pltpu.make_async_copy(pre_hbm.at[r + 1], b1, s1).wait()
        process(b1)
        return 0

    lax.fori_loop(0, _ROWS_PT // 2, pair, 0)
    pltpu.sync_copy(ha, out_hbm.at[2 * wid])
    pltpu.sync_copy(hb, out_hbm.at[2 * wid + 1])


@functools.lru_cache(maxsize=None)
def _make_hist_hi():
    @functools.partial(
        pl.kernel,
        out_type=jax.ShapeDtypeStruct((2 * _NTILES, _NB1), jnp.int32),
        mesh=_sc_mesh(),
        compiler_params=pltpu.CompilerParams(needs_layout_passes=False),
        scratch_types=[
            pltpu.VMEM((_D_SAE,), jnp.float32),
            pltpu.VMEM((_D_SAE,), jnp.float32),
            pltpu.VMEM((_NB1,), jnp.int32),
            pltpu.VMEM((_NB1,), jnp.int32),
            pltpu.SemaphoreType.DMA,
            pltpu.SemaphoreType.DMA,
        ],
    )
    def _hist_hi(pre_hbm, out_hbm, b0, b1, ha, hb, s0, s1):
        wid = _tile_wid()

        def binfn(bits):
            hi = lax.shift_right_logical(bits, 16)
            return hi, hi > 0

        _hist_pass(pre_hbm, out_hbm, b0, b1, ha, hb, s0, s1, wid, binfn)

    return _hist_hi


@functools.lru_cache(maxsize=None)
def _make_hist_lo():
    @functools.partial(
        pl.kernel,
        out_type=jax.ShapeDtypeStruct((2 * _NTILES, _NB2), jnp.int32),
        mesh=_sc_mesh(),
        compiler_params=pltpu.CompilerParams(needs_layout_passes=False),
        scratch_types=[
            pltpu.VMEM((_D_SAE,), jnp.float32),
            pltpu.VMEM((_D_SAE,), jnp.float32),
            pltpu.VMEM((_NB2,), jnp.int32),
            pltpu.VMEM((_NB2,), jnp.int32),
            pltpu.VMEM((16,), jnp.int32),
            pltpu.SemaphoreType.DMA,
            pltpu.SemaphoreType.DMA,
        ],
    )
    def _hist_lo(pre_hbm, sel_hbm, out_hbm, b0, b1, ha, hb, selv, s0, s1):
        wid = _tile_wid()
        pltpu.sync_copy(sel_hbm, selv)
        sel = selv[...]
        m15 = jnp.full((16,), 0x7FFF, jnp.int32)

        def binfn(bits):
            hi = lax.shift_right_logical(bits, 16)
            mid = jnp.bitwise_and(lax.shift_right_logical(bits, 1), m15)
            return mid, hi == sel

        _hist_pass(pre_hbm, out_hbm, b0, b1, ha, hb, s0, s1, wid, binfn)

    return _hist_lo


def _scan_body(k_ref, h_ref, b_ref, a_ref):
    h = jnp.sum(h_ref[...], axis=0)
    hf = h.astype(jnp.float32)
    r, c = h.shape
    ci = lax.broadcasted_iota(jnp.int32, (c, c), 0)
    cj = lax.broadcasted_iota(jnp.int32, (c, c), 1)
    tri = (ci >= cj).astype(jnp.float32)
    rs = jax.lax.dot(hf, tri, precision=jax.lax.Precision.HIGHEST)
    ri = lax.broadcasted_iota(jnp.int32, (r, r), 0)
    rj = lax.broadcasted_iota(jnp.int32, (r, r), 1)
    upp = (rj > ri).astype(jnp.float32)
    t = rs[:, 0:1]
    sr = jax.lax.dot(upp, t, precision=jax.lax.Precision.HIGHEST)
    s = rs + sr
    kf = k_ref[0, 0].astype(jnp.float32)
    nkeep = jnp.sum((s >= kf).astype(jnp.int32))
    bstar = nkeep - 1
    flat = (lax.broadcasted_iota(jnp.int32, (r, c), 0) * c
            + lax.broadcasted_iota(jnp.int32, (r, c), 1))
    above = jnp.sum(jnp.where(flat > bstar, h, 0))
    b_ref[0, 0] = bstar
    a_ref[0, 0] = above


def _scan(hists3d, k11):
    nt, r, c = hists3d.shape
    return pl.pallas_call(
        _scan_body,
        in_specs=[
            pl.BlockSpec(memory_space=pltpu.SMEM),
            pl.BlockSpec((nt, r, c), lambda: (0, 0, 0)),
        ],
        out_specs=[
            pl.BlockSpec(memory_space=pltpu.SMEM),
            pl.BlockSpec(memory_space=pltpu.SMEM),
        ],
        out_shape=[
            jax.ShapeDtypeStruct((1, 1), jnp.int32),
            jax.ShapeDtypeStruct((1, 1), jnp.int32),
        ],
    )(k11, hists3d)


_BM_D = 1024
_BK_D = 1536


def _dec_body(tau_ref, pre_ref, wd_ref, bd_ref, z_ref, rec_ref):
    j = pl.program_id(1)
    tau = tau_ref[0, 0]
    p = pre_ref[...]
    z = jnp.where(p >= tau, p, 0.0)
    z_ref[...] = z
    zb = z.astype(jnp.bfloat16)
    contrib = jnp.dot(zb, wd_ref[...], preferred_element_type=jnp.float32)

    @pl.when(j == 0)
    def _():
        rec_ref[...] = contrib + bd_ref[...]

    @pl.when(j > 0)
    def _():
        rec_ref[...] += contrib


def _decode(tau11, pre, W_decb, b_dec2d):
    grid = (_B // _BM_D, _D_SAE // _BK_D)
    return pl.pallas_call(
        _dec_body,
        grid=grid,
        in_specs=[
            pl.BlockSpec(memory_space=pltpu.SMEM),
            pl.BlockSpec((_BM_D, _BK_D), lambda i, j: (i, j)),
            pl.BlockSpec((_BK_D, _D_IN), lambda i, j: (j, 0)),
            pl.BlockSpec((1, _D_IN), lambda i, j: (0, 0)),
        ],
        out_specs=[
            pl.BlockSpec((_BM_D, _BK_D), lambda i, j: (i, j)),
            pl.BlockSpec((_BM_D, _D_IN), lambda i, j: (i, 0)),
        ],
        out_shape=[
            jax.ShapeDtypeStruct((_B, _D_SAE), jnp.float32),
            jax.ShapeDtypeStruct((_B, _D_IN), jnp.float32),
        ],
    )(tau11, pre, W_decb, b_dec2d)


def kernel(x, W_enc, b_enc, W_dec, b_dec):
    xcb = (x - b_dec[None, :]).astype(jnp.bfloat16)
    W_encb = W_enc.astype(jnp.bfloat16)
    W_decb = W_dec.astype(jnp.bfloat16)
    b_enc2d = b_enc.reshape(1, _D_SAE)
    b_dec2d = b_dec.reshape(1, _D_IN)

    pre = _encode(xcb, W_encb, b_enc2d)

    h1 = _make_hist_hi()(pre)
    k11 = jnp.full((1, 1), _K_TOTAL, jnp.int32)
    b1, a1 = _scan(h1.reshape(2 * _NTILES, _NB1 // 128, 128), k11)

    sel16 = jnp.broadcast_to(b1.reshape(1), (16,)).astype(jnp.int32)
    h2 = _make_hist_lo()(pre, sel16)
    r11 = k11 - a1
    b2, _ = _scan(h2.reshape(2 * _NTILES, _NB2 // 128, 128), r11)

    tau_bits = jnp.bitwise_or(jnp.left_shift(b1, 16), jnp.left_shift(b2, 1))
    tau11 = jax.lax.bitcast_convert_type(tau_bits, jnp.float32)

    z, recon = _decode(tau11, pre, W_decb, b_dec2d)
    return (recon, z)
